# baseline (speedup 1.0000x reference)
import jax
import jax.numpy as jnp
from jax import lax
from jax.experimental import pallas as pl
from jax.experimental.pallas import tpu as pltpu
from jax.experimental.pallas import tpu_sc as plsc

_N = 500000
_C = 100
_NB = 15
_L = 16
_NW = 32
_CH = 256
_K = 38
_CW = _K * _CH
_S = _NW * _CW
_BC = 2048
_TCG = -(-(_N - _S) // _BC)


def _sc_body(prob_hbm, raw_hbm, out_hbm, raw_v, buf_a, buf_b, acc_c, acc_a,
             stage, sem_a, sem_b):
    wid = lax.axis_index("c") * 16 + lax.axis_index("s")
    base = pl.multiple_of(wid * _CW, _CH)

    iota = lax.iota(jnp.int32, _L)
    zeros_f = jnp.zeros((_L,), jnp.float32)

    pltpu.sync_copy(raw_hbm.at[pl.ds(2 * base, 2 * _CW)], raw_v)

    for k in range(_L):
        acc_c[pl.ds(k * _L, _L)] = zeros_f
        acc_a[pl.ds(k * _L, _L)] = zeros_f

    def start(c, buf, sem):
        col0 = pl.multiple_of(base + c * _CH, _CH)
        return pltpu.async_copy(prob_hbm.at[:, pl.ds(col0, _CH)], buf, sem)

    def wait(buf, sem):
        pltpu.make_async_copy(prob_hbm.at[:, pl.ds(0, _CH)], buf, sem).wait()

    def process(off0, buf):
        for q in range(_CH // _L):
            lc = q * _L + iota
            off = off0 + q * _L
            blk = off >> 7
            a0 = blk * 256 + (off & 127)
            labels = raw_v[pl.ds(a0, _L)]
            preds = raw_v[pl.ds(a0 + 128, _L)]
            conf = plsc.load_gather(buf, [preds, lc])
            acc = jnp.where(preds == labels, 1.0, 0.0).astype(jnp.float32)
            t = conf * jnp.float32(_NB)
            ti = t.astype(jnp.int32)
            onedge = t == ti.astype(jnp.float32)
            b = ti - jnp.where(onedge, 1, 0)
            b = jnp.where(conf <= 0.0, _NB, b)
            b = jnp.minimum(jnp.maximum(b, 0), _NB)
            addr = iota * _L + b
            plsc.addupdate_scatter(acc_c, [addr], conf)
            plsc.addupdate_scatter(acc_a, [addr], acc)

    start(0, buf_a, sem_a)

    def pair(p, _):
        ca = 2 * p
        wait(buf_a, sem_a)
        start(ca + 1, buf_b, sem_b)
        process(ca * _CH, buf_a)
        wait(buf_b, sem_b)
        start(ca + 2, buf_a, sem_a)
        process((ca + 1) * _CH, buf_b)
        return _

    lax.fori_loop(0, _K // 2 - 1, pair, None)

    wait(buf_a, sem_a)
    start(_K - 1, buf_b, sem_b)
    process((_K - 2) * _CH, buf_a)
    wait(buf_b, sem_b)
    process((_K - 1) * _CH, buf_b)

    cs = acc_c[pl.ds(0, _L)]
    as_ = acc_a[pl.ds(0, _L)]
    for l in range(1, _L):
        cs = cs + acc_c[pl.ds(l * _L, _L)]
        as_ = as_ + acc_a[pl.ds(l * _L, _L)]
    stage[pl.ds(0, _L)] = cs
    stage[pl.ds(_L, _L)] = as_
    pltpu.sync_copy(stage, out_hbm.at[wid])


def _tc_body(pk_ref, prob_ref, out_ref):
    g = pl.program_id(0)
    pk = pk_ref[...]
    prob = prob_ref[...]
    preds = pk & 127
    accb = (pk >> 7).astype(jnp.float32)
    rows = lax.broadcasted_iota(jnp.int32, (_C, _BC), 0)
    conf = jnp.sum(jnp.where(rows == preds[None, :], prob, 0.0), axis=0)
    col = _S + g * _BC + lax.broadcasted_iota(jnp.int32, (_BC,), 0)
    valid = col < _N
    t = conf * jnp.float32(_NB)
    ti = t.astype(jnp.int32)
    onedge = t == ti.astype(jnp.float32)
    b = ti - jnp.where(onedge, 1, 0)
    b = jnp.where(conf <= 0.0, _NB, b)
    b = jnp.where(valid, b, _NB)
    b = jnp.minimum(jnp.maximum(b, 0), _NB)
    bins = lax.broadcasted_iota(jnp.int32, (_L, _BC), 0)
    sel = bins == b[None, :]
    conf_sums = jnp.sum(jnp.where(sel, conf[None, :], 0.0), axis=1)
    acc_sums = jnp.sum(jnp.where(sel, accb[None, :], 0.0), axis=1)

    @pl.when(g == 0)
    def _():
        out_ref[...] = jnp.zeros((2, _L), jnp.float32)

    out_ref[...] += jnp.stack([conf_sums, acc_sums])


@jax.jit
def _ece_both(prob_t, raw_tp, packed_tc):
    mesh = plsc.VectorSubcoreMesh(core_axis_name="c", subcore_axis_name="s")
    sc_partials = pl.kernel(
        _sc_body,
        out_type=jax.ShapeDtypeStruct((_NW, 2 * _L), jnp.float32),
        mesh=mesh,
        compiler_params=pltpu.CompilerParams(needs_layout_passes=False,
                                             use_tc_tiling_on_sc=True),
        scratch_types=[
            pltpu.VMEM((2 * _CW,), jnp.int32),
            pltpu.VMEM((_C, _CH), jnp.float32),
            pltpu.VMEM((_C, _CH), jnp.float32),
            pltpu.VMEM((_L * _L,), jnp.float32),
            pltpu.VMEM((_L * _L,), jnp.float32),
            pltpu.VMEM((2 * _L,), jnp.float32),
            pltpu.SemaphoreType.DMA,
            pltpu.SemaphoreType.DMA,
        ],
    )(prob_t, raw_tp)

    tc_partials = pl.pallas_call(
        _tc_body,
        grid=(_TCG,),
        in_specs=[
            pl.BlockSpec((_BC,), lambda g: (g,)),
            pl.BlockSpec((_C, _BC), lambda g: (0, _S // _BC + g)),
        ],
        out_specs=pl.BlockSpec((2, _L), lambda g: (0, 0)),
        out_shape=jax.ShapeDtypeStruct((2, _L), jnp.float32),
    )(packed_tc, prob_t)

    return sc_partials, tc_partials


def kernel(probabilities, target_pred):
    tp_pad = jnp.pad(target_pred, ((0, 500096 - _N), (0, 0)))
    raw_tp = tp_pad.T.reshape(2, 3907, 128).transpose(1, 0, 2).reshape(-1)
    labels = target_pred[_S:, 0]
    preds = target_pred[_S:, 1]
    packed_tc = jnp.where(labels == preds, 128, 0) | preds
    sc_partials, tc_partials = _ece_both(probabilities.T, raw_tp, packed_tc)
    tot = sc_partials.sum(axis=0).reshape(2, _L) + tc_partials
    diff = jnp.abs(tot[1, :_NB] - tot[0, :_NB])
    return diff.sum() / jnp.float32(_N)


# speedup vs baseline: 1.0788x; 1.0788x over previous
import jax
import jax.numpy as jnp
from jax import lax
from jax.experimental import pallas as pl
from jax.experimental.pallas import tpu as pltpu
from jax.experimental.pallas import tpu_sc as plsc

_N = 500000
_C = 100
_NB = 15
_L = 16
_NW = 32
_CH = 256
_K = 36
_CW = _K * _CH
_S = _NW * _CW
_BC = 2048
_TCG = -(-(_N - _S) // _BC)


def _sc_body(prob_hbm, pk_hbm, out_hbm, pk_v, buf_a, buf_b, acc_c, acc_a,
             stage, sem_a, sem_b):
    wid = lax.axis_index("c") * 16 + lax.axis_index("s")
    base = pl.multiple_of(wid * _CW, _CH)

    iota = lax.iota(jnp.int32, _L)
    zeros_f = jnp.zeros((_L,), jnp.float32)

    pltpu.sync_copy(pk_hbm.at[pl.ds(base, _CW)], pk_v)

    for k in range(_L):
        acc_c[pl.ds(k * _L, _L)] = zeros_f
        acc_a[pl.ds(k * _L, _L)] = zeros_f

    def start(c, buf, sem):
        col0 = pl.multiple_of(base + c * _CH, _CH)
        return pltpu.async_copy(prob_hbm.at[:, pl.ds(col0, _CH)], buf, sem)

    def wait(buf, sem):
        pltpu.make_async_copy(prob_hbm.at[:, pl.ds(0, _CH)], buf, sem).wait()

    def process(off0, buf):
        for q in range(_CH // _L):
            lc = q * _L + iota
            off = off0 + q * _L
            pk = pk_v[pl.ds(off, _L)]
            preds = pk & 127
            conf = plsc.load_gather(buf, [preds, lc])
            acc = (pk >> 7).astype(jnp.float32)
            t = conf * jnp.float32(_NB)
            ti = t.astype(jnp.int32)
            onedge = t == ti.astype(jnp.float32)
            b = ti - jnp.where(onedge, 1, 0)
            b = jnp.where(conf <= 0.0, _NB, b)
            b = jnp.minimum(jnp.maximum(b, 0), _NB)
            addr = iota * _L + b
            plsc.addupdate_scatter(acc_c, [addr], conf)
            plsc.addupdate_scatter(acc_a, [addr], acc)

    start(0, buf_a, sem_a)

    def pair(p, _):
        ca = 2 * p
        wait(buf_a, sem_a)
        start(ca + 1, buf_b, sem_b)
        process(ca * _CH, buf_a)
        wait(buf_b, sem_b)
        start(ca + 2, buf_a, sem_a)
        process((ca + 1) * _CH, buf_b)
        return _

    lax.fori_loop(0, _K // 2 - 1, pair, None)

    wait(buf_a, sem_a)
    start(_K - 1, buf_b, sem_b)
    process((_K - 2) * _CH, buf_a)
    wait(buf_b, sem_b)
    process((_K - 1) * _CH, buf_b)

    cs = acc_c[pl.ds(0, _L)]
    as_ = acc_a[pl.ds(0, _L)]
    for l in range(1, _L):
        cs = cs + acc_c[pl.ds(l * _L, _L)]
        as_ = as_ + acc_a[pl.ds(l * _L, _L)]
    stage[pl.ds(0, _L)] = cs
    stage[pl.ds(_L, _L)] = as_
    pltpu.sync_copy(stage, out_hbm.at[wid])


def _tc_body(pk_ref, prob_ref, out_ref):
    g = pl.program_id(0)
    pk = pk_ref[...]
    prob = prob_ref[...]
    preds = pk & 127
    accb = (pk >> 7).astype(jnp.float32)
    rows = lax.broadcasted_iota(jnp.int32, (_C, _BC), 0)
    conf = jnp.sum(jnp.where(rows == preds[None, :], prob, 0.0), axis=0)
    col = _S + g * _BC + lax.broadcasted_iota(jnp.int32, (_BC,), 0)
    valid = col < _N
    t = conf * jnp.float32(_NB)
    ti = t.astype(jnp.int32)
    onedge = t == ti.astype(jnp.float32)
    b = ti - jnp.where(onedge, 1, 0)
    b = jnp.where(conf <= 0.0, _NB, b)
    b = jnp.where(valid, b, _NB)
    b = jnp.minimum(jnp.maximum(b, 0), _NB)
    bins = lax.broadcasted_iota(jnp.int32, (_L, _BC), 0)
    sel = bins == b[None, :]
    conf_sums = jnp.sum(jnp.where(sel, conf[None, :], 0.0), axis=1)
    acc_sums = jnp.sum(jnp.where(sel, accb[None, :], 0.0), axis=1)

    @pl.when(g == 0)
    def _():
        out_ref[...] = jnp.zeros((2, _L), jnp.float32)

    out_ref[...] += jnp.stack([conf_sums, acc_sums])


@jax.jit
def _ece_both(prob_t, packed):
    mesh = plsc.VectorSubcoreMesh(core_axis_name="c", subcore_axis_name="s")
    sc_partials = pl.kernel(
        _sc_body,
        out_type=jax.ShapeDtypeStruct((_NW, 2 * _L), jnp.float32),
        mesh=mesh,
        compiler_params=pltpu.CompilerParams(needs_layout_passes=False,
                                             use_tc_tiling_on_sc=True),
        scratch_types=[
            pltpu.VMEM((_CW,), jnp.int32),
            pltpu.VMEM((_C, _CH), jnp.float32),
            pltpu.VMEM((_C, _CH), jnp.float32),
            pltpu.VMEM((_L * _L,), jnp.float32),
            pltpu.VMEM((_L * _L,), jnp.float32),
            pltpu.VMEM((2 * _L,), jnp.float32),
            pltpu.SemaphoreType.DMA,
            pltpu.SemaphoreType.DMA,
        ],
    )(prob_t, packed)

    tc_partials = pl.pallas_call(
        _tc_body,
        grid=(_TCG,),
        in_specs=[
            pl.BlockSpec((_BC,), lambda g: (_S // _BC + g,)),
            pl.BlockSpec((_C, _BC), lambda g: (0, _S // _BC + g)),
        ],
        out_specs=pl.BlockSpec((2, _L), lambda g: (0, 0)),
        out_shape=jax.ShapeDtypeStruct((2, _L), jnp.float32),
    )(packed, prob_t)

    return sc_partials, tc_partials


def kernel(probabilities, target_pred):
    labels = target_pred[:, 0]
    preds = target_pred[:, 1]
    packed = jnp.where(labels == preds, 128, 0) | preds
    sc_partials, tc_partials = _ece_both(probabilities.T, packed)
    tot = sc_partials.sum(axis=0).reshape(2, _L) + tc_partials
    diff = jnp.abs(tot[1, :_NB] - tot[0, :_NB])
    return diff.sum() / jnp.float32(_N)
